# baseline (speedup 1.0000x reference)
import functools

import jax
import jax.numpy as jnp
from jax import lax
from jax.experimental import pallas as pl
from jax.experimental.pallas import tpu as pltpu
from jax.experimental.pallas import tpu_sc as plsc

N = 10000
E = 320000
D = 128
NPAD = 10240
NTILES = 32
NPT = NPAD // NTILES
CHUNK = 6400
K = 64
PEND = CHUNK + 2 * K
NEG = -3.0e38
SENT = -1.0e30


def _enc_body(x_ref, w1_ref, bz_ref, wa_ref, wb_ref, z_ref, a_ref, b_ref):
    z = jnp.dot(x_ref[...], w1_ref[...], preferred_element_type=jnp.float32)
    z = z + bz_ref[...]
    z_ref[...] = z
    a_ref[...] = jnp.dot(z, wa_ref[...], preferred_element_type=jnp.float32)
    b_ref[...] = jnp.dot(z, wb_ref[...], preferred_element_type=jnp.float32)


def _encode(x_p, W1, bz, Wa, Wb):
    BLK = 1024
    grid = (NPAD // BLK,)
    row_spec = pl.BlockSpec((BLK, D), lambda i: (i, 0))
    w_spec = pl.BlockSpec((D, D), lambda i: (0, 0))
    b_spec = pl.BlockSpec((1, D), lambda i: (0, 0))
    out = jax.ShapeDtypeStruct((NPAD, D), jnp.float32)
    return pl.pallas_call(
        _enc_body,
        grid=grid,
        in_specs=[row_spec, w_spec, b_spec, w_spec, w_spec],
        out_specs=[row_spec, row_spec, row_spec],
        out_shape=[out, out, out],
    )(x_p, W1, bz, Wa, Wb)


def _upd_body(z_ref, a_ref, s_ref, bm_ref, u1a_ref, u1b_ref, bu1_ref,
              wu2_ref, bu2_ref, wda_ref, wdb_ref, bd_ref, y_ref, h_ref):
    z = z_ref[...]
    seg = s_ref[...]
    agg = jnp.where(seg > SENT, a_ref[...] + seg + bm_ref[...], 0.0)
    t = jnp.dot(z, u1a_ref[...], preferred_element_type=jnp.float32)
    t = t + jnp.dot(agg, u1b_ref[...], preferred_element_type=jnp.float32)
    t = jnp.maximum(t + bu1_ref[...], 0.0)
    h = jnp.dot(t, wu2_ref[...], preferred_element_type=jnp.float32) + bu2_ref[...]
    h_ref[...] = h
    y = jnp.dot(z, wda_ref[...], preferred_element_type=jnp.float32)
    y = y + jnp.dot(h, wdb_ref[...], preferred_element_type=jnp.float32)
    y_ref[...] = y + bd_ref[...]


def _update(z, A, seg, bm, U1a, U1b, bu1, Wu2, bu2, Wda, Wdb, bd):
    BLK = 1024
    grid = (NPAD // BLK,)
    row_spec = pl.BlockSpec((BLK, D), lambda i: (i, 0))
    w_spec = pl.BlockSpec((D, D), lambda i: (0, 0))
    b_spec = pl.BlockSpec((1, D), lambda i: (0, 0))
    out = jax.ShapeDtypeStruct((NPAD, D), jnp.float32)
    return pl.pallas_call(
        _upd_body,
        grid=grid,
        in_specs=[row_spec, row_spec, row_spec, b_spec, w_spec, w_spec,
                  b_spec, w_spec, b_spec, w_spec, w_spec, b_spec],
        out_specs=[row_spec, row_spec],
        out_shape=[out, out],
    )(z, A, seg, bm, U1a, U1b, bu1, Wu2, bu2, Wda, Wdb, bd)


EPT = E // NTILES
VPT = EPT // 16
NBURST = (VPT + 7) // 8
VBUF = NBURST * 128
EPAD = NTILES * EPT + 128
CAP = 10240
BT = NTILES * NTILES * CAP
BSZ = BT + 128


def _bin_body(u_hbm, v_hbm, w_hbm, bu_hbm, bv_hbm, bw_hbm, cnts_hbm,
              ubuf, vbuf, wbuf, dstbuf, countv, cidx, cnt_smem, sem):
    c = lax.axis_index("c")
    s = lax.axis_index("s")
    tid = s * 2 + c

    pltpu.sync_copy(u_hbm.at[pl.ds(tid * EPT, VBUF)], ubuf)
    pltpu.sync_copy(v_hbm.at[pl.ds(tid * EPT, VBUF)], vbuf)
    pltpu.sync_copy(w_hbm.at[pl.ds(tid * EPT, VBUF)], wbuf)

    for b in range(NTILES):
        cnt_smem[b] = 0

    iota = lax.iota(jnp.int32, 16)

    def burst_body(bi, carry):
        for j in range(8):
            g = bi * 8 + j
            uvec = ubuf[pl.ds(g * 16, 16)]
            bvec = jnp.clip(
                lax.shift_right_arithmetic(uvec * 6554, 21), 0, NTILES - 1)
            valid = g < VPT
            inc = valid.astype(jnp.int32)
            dvec = jnp.zeros((16,), jnp.int32)
            for k in range(16):
                b = bvec[k]
                cur = cnt_smem[b]
                cnt_smem[b] = cur + inc
                dst = (b * NTILES + tid) * CAP + cur
                dvec = jnp.where(iota == k, dst, dvec)
            dump = BT + j * 16 + iota
            dvec = jnp.where(valid, dvec, dump)
            dstbuf[pl.ds(j * 16, 16)] = dvec
        src = bi * 128
        cp1 = pltpu.async_copy(ubuf.at[pl.ds(src, 128)], bu_hbm.at[dstbuf], sem)
        cp2 = pltpu.async_copy(vbuf.at[pl.ds(src, 128)], bv_hbm.at[dstbuf], sem)
        cp3 = pltpu.async_copy(wbuf.at[pl.ds(src, 128)], bw_hbm.at[dstbuf], sem)
        cp1.wait()
        cp2.wait()
        cp3.wait()
        return carry

    lax.fori_loop(0, NBURST, burst_body, 0)

    cv0 = jnp.zeros((16,), jnp.int32)
    cv1 = jnp.zeros((16,), jnp.int32)
    for b in range(16):
        cv0 = jnp.where(iota == b, cnt_smem[b], cv0)
        cv1 = jnp.where(iota == b, cnt_smem[b + 16], cv1)
    countv[pl.ds(0, 16)] = cv0
    countv[pl.ds(16, 16)] = cv1
    cidx[pl.ds(0, 16)] = iota * NTILES + tid
    cidx[pl.ds(16, 16)] = (iota + 16) * NTILES + tid
    pltpu.async_copy(countv, cnts_hbm.at[cidx], sem).wait()


def _agg_body(bu_hbm, bv_hbm, bw_hbm, cnts_hbm, bt_hbm, r_hbm, seg_hbm,
              cbuf, ub, wb, idxb, rows, acc, rbuf, cnt_smem, sem):
    c = lax.axis_index("c")
    s = lax.axis_index("s")
    wid = s * 2 + c
    base = wid * NPT

    pltpu.sync_copy(r_hbm, rbuf)
    pltpu.sync_copy(cnts_hbm, cbuf)

    negv = jnp.full((16,), NEG, jnp.float32)

    def init_body(i, carry):
        acc[pl.ds(i * 16, 16)] = negv
        return carry

    lax.fori_loop(0, (NPT + 1) * D // 16, init_body, 0)

    iota = lax.iota(jnp.int32, 16)
    rks = [rbuf[pl.ds(f * 16, 16)] for f in range(D // 16)]
    cvec0 = cbuf[pl.ds(wid * NTILES, 16)]
    cvec1 = cbuf[pl.ds(wid * NTILES + 16, 16)]
    for t in range(16):
        cnt_smem[t] = cvec0[t]
        cnt_smem[t + 16] = cvec1[t]

    def t_body(t, carry0):
        cnt_tb = cnt_smem[t]
        region = (wid * NTILES + t) * CAP
        nch = (cnt_tb + 127) // 128

        def ch_body(ch, carry):
            off = region + ch * 128
            pltpu.sync_copy(bu_hbm.at[pl.ds(off, 128)], ub)
            pltpu.sync_copy(bw_hbm.at[pl.ds(off, 128)], wb)
            pltpu.sync_copy(bv_hbm.at[pl.ds(off, 128)], idxb)
            for g8 in range(8):
                vv = idxb[pl.ds(g8 * 16, 16)]
                idxb[pl.ds(g8 * 16, 16)] = jnp.clip(vv, 0, NPAD - 1)
            gcp = pltpu.async_copy(bt_hbm.at[idxb], rows, sem)
            gcp.wait()
            ne = jnp.minimum(cnt_tb - ch * 128, 128)

            def grp_body(g, carry2):
                lvec = iota + g * 16
                uvec = ub[pl.ds(g * 16, 16)]
                u_eff = jnp.where(lvec < ne, uvec, base + NPT)
                lu = u_eff - base
                wv = wb[pl.ds(g * 16, 16)]
                for k2 in range(16):
                    luk = lu[k2]
                    wk = wv[k2]
                    ao = luk * D
                    j = g * 16 + k2
                    for f in range(D // 16):
                        a = acc[pl.ds(ao + f * 16, 16)]
                        m = rows[j, pl.ds(f * 16, 16)]
                        acc[pl.ds(ao + f * 16, 16)] = jnp.maximum(
                            a, m + wk * rks[f])
                return carry2

            lax.fori_loop(0, 8, grp_body, 0)
            return carry

        lax.fori_loop(0, nch, ch_body, 0)
        return carry0

    lax.fori_loop(0, NTILES, t_body, 0)

    pltpu.sync_copy(acc.at[pl.ds(0, NPT * D)],
                    seg_hbm.at[pl.ds(wid * NPT * D, NPT * D)])


def _segmax(u, v, w, B, r):
    mesh = plsc.VectorSubcoreMesh(core_axis_name="c", subcore_axis_name="s")
    binfn = functools.partial(
        pl.kernel,
        mesh=mesh,
        out_type=[
            jax.ShapeDtypeStruct((BSZ,), jnp.int32),
            jax.ShapeDtypeStruct((BSZ,), jnp.int32),
            jax.ShapeDtypeStruct((BSZ,), jnp.float32),
            jax.ShapeDtypeStruct((NTILES * NTILES,), jnp.int32),
        ],
        scratch_types=[
            pltpu.VMEM((VBUF,), jnp.int32),
            pltpu.VMEM((VBUF,), jnp.int32),
            pltpu.VMEM((VBUF,), jnp.float32),
            pltpu.VMEM((128,), jnp.int32),
            pltpu.VMEM((NTILES,), jnp.int32),
            pltpu.VMEM((NTILES,), jnp.int32),
            pltpu.SMEM((NTILES,), jnp.int32),
            pltpu.SemaphoreType.DMA,
        ],
    )(_bin_body)
    bu, bv, bw, cnts = binfn(u, v, w)

    aggfn = functools.partial(
        pl.kernel,
        mesh=mesh,
        out_type=jax.ShapeDtypeStruct((NPAD * D,), jnp.float32),
        scratch_types=[
            pltpu.VMEM((NTILES * NTILES,), jnp.int32),
            pltpu.VMEM((128,), jnp.int32),
            pltpu.VMEM((128,), jnp.float32),
            pltpu.VMEM((128,), jnp.int32),
            pltpu.VMEM((128, D), jnp.float32),
            pltpu.VMEM(((NPT + 1) * D,), jnp.float32),
            pltpu.VMEM((D,), jnp.float32),
            pltpu.SMEM((NTILES,), jnp.int32),
            pltpu.SemaphoreType.DMA,
        ],
    )(_agg_body)
    return aggfn(bu, bv, bw, cnts, B, r)


def kernel(x, edge_index, edge_weight, W_enc, b_enc, W_msg, b_msg,
           W_u1, b_u1, W_u2, b_u2, W_dec, b_dec):
    x_p = jnp.pad(x, ((0, NPAD - N), (0, 0)))
    u = jnp.pad(edge_index[0].astype(jnp.int32), (0, EPAD - E))
    v = jnp.pad(edge_index[1].astype(jnp.int32), (0, EPAD - E))
    w = jnp.pad(edge_weight.astype(jnp.float32), (0, EPAD - E))

    W1 = W_enc[:D]
    Wa = W_msg[:D]
    Wb = W_msg[D:2 * D]
    r = W_msg[2 * D]

    z, A, B = _encode(x_p, W1, b_enc.reshape(1, D), Wa, Wb)
    seg = _segmax(u, v, w, B, r).reshape(NPAD, D)
    y, h = _update(z, A, seg, b_msg.reshape(1, D),
                   W_u1[:D], W_u1[D:], b_u1.reshape(1, D),
                   W_u2, b_u2.reshape(1, D),
                   W_dec[:D], W_dec[D:], b_dec.reshape(1, D))
    return (y[:N], h[:N])
